# baseline (speedup 1.0000x reference)
import jax
import jax.numpy as jnp
from jax import lax
from jax.experimental import pallas as pl
from jax.experimental.pallas import tpu as pltpu
from jax.experimental.pallas import tpu_sc as plsc

F = 26
V = 100000
K = 16
B = 16384

NC = 2
NS = 16
NW = NC * NS

WC = 128
CPF = V // WC
TAIL = V - CPF * WC
RPF = 12500
VP = RPF * 8
SC_ROWS = F * RPF
NFULL = F * CPF
NB = 8
NITER = 80
OBN = WC * 16

ROWS_PER_W = B // NW
BC = 128
STEPS = ROWS_PER_W // BC
GROUPS = BC // 16
IDX_PER_CHUNK = BC * F
IDX_ROWS = IDX_PER_CHUNK // 128
IDX_ROWS_PAD = 32
N_CHUNKS = NW * STEPS


def _tr_body(src, tailsrc, out, *scr):
    slabs = scr[0:NB]
    obufs = scr[NB:2 * NB]
    semi = scr[2 * NB:3 * NB]
    semo = scr[3 * NB:4 * NB]
    cid = lax.axis_index("c")
    sid = lax.axis_index("s")
    w = cid * NS + sid
    lanes = lax.iota(jnp.int32, 16)

    def fc(j):
        u = w + NW * j
        return u, u // CPF, u % CPF

    def start_in(b, j):
        u, f, c = fc(j)

        @pl.when(u < NFULL)
        def _():
            pltpu.async_copy(
                src.at[pl.ds(f * 16, 16), pl.ds(c * WC, WC)],
                slabs[b], semi[b])

    def wait_in(b):
        pltpu.make_async_copy(
            src.at[pl.ds(0, 16), pl.ds(0, WC)], slabs[b], semi[b]).wait()

    lanes16 = lanes * 16
    idxr = [lanes16 + r for r in range(8)]

    def transpose_slab(b, ncols):
        for k in range(16):
            nch = ncols // 16
            vecs = [slabs[b][k, pl.ds(ch * 16, 16)] for ch in range(nch)]
            for ch in range(nch):
                off = ch * 256 + (k // 8) * 8
                plsc.store_scatter(
                    obufs[b].at[pl.ds(off, OBN - off)], [idxr[k % 8]],
                    vecs[ch])

    def start_out(b, j):
        u, f, c = fc(j)
        pltpu.async_copy(
            obufs[b], out.at[pl.ds((f * RPF + c * (WC // 8)) * 128, OBN)],
            semo[b])

    def wait_out(b):
        pltpu.make_async_copy(
            obufs[b], out.at[pl.ds(0, OBN)], semo[b]).wait()

    for b in range(NB):
        start_in(b, b)

    def ring(i, carry):
        for b in range(NB):
            j = NB * i + b
            u = w + NW * j

            @pl.when(jnp.logical_and(j >= NB, w + NW * (j - NB) < NFULL))
            def _():
                wait_out(b)

            @pl.when(u < NFULL)
            def _():
                wait_in(b)
                transpose_slab(b, WC)
                start_out(b, j)

            start_in(b, j + NB)
        return carry

    lax.fori_loop(0, NITER, ring, 0)

    for b in range(NB):
        jf = NB * (NITER - 1) + b

        @pl.when(w + NW * jf < NFULL)
        def _():
            wait_out(b)

    @pl.when(w < F)
    def _():
        pltpu.async_copy(
            tailsrc.at[pl.ds(w * 16, 16)], slabs[0], semi[0]).wait()
        transpose_slab(0, TAIL)
        pltpu.async_copy(
            obufs[0].at[pl.ds(0, TAIL * 16)],
            out.at[pl.ds((w * RPF + CPF * (WC // 8)) * 128, TAIL * 16)],
            semo[0]).wait()


def _fm_body(xw, emb, wtab, out, idx_w, ebuf, wbuf, obuf, sem):
    cid = lax.axis_index("c")
    sid = lax.axis_index("s")
    wid = cid * NS + sid

    lanes = lax.iota(jnp.int32, 16)

    def step_fn(step, carry):
        row0 = wid * ROWS_PER_W + step * BC
        chunk = wid * STEPS + step
        pltpu.sync_copy(xw.at[chunk], idx_w)
        copies = []
        for j in range(IDX_ROWS):
            copies.append(pltpu.async_copy(
                emb.at[idx_w.at[j]], ebuf.at[pl.ds(j * 128, 128)], sem))
            copies.append(pltpu.async_copy(
                wtab.at[idx_w.at[j]], wbuf.at[pl.ds(j * 128, 128)], sem))
        for c in copies:
            c.wait()

        def group_fn(g, gcarry):
            goff = g * 16
            lin = wbuf[pl.ds(goff, 16)]
            for f in range(1, F):
                lin = lin + wbuf[pl.ds(f * BC + goff, 16)]
            inter = jnp.zeros((16,), jnp.float32)
            for r16 in range(16):
                roff = goff + r16
                e = ebuf[roff, :]
                s = e
                q = e * e
                for f in range(1, F):
                    e = ebuf[f * BC + roff, :]
                    s = s + e
                    q = q + e * e
                t = s * s - q
                tot = t[0]
                for i in range(1, 16):
                    tot = tot + t[i]
                inter = jnp.where(lanes == r16, tot, inter)
            z = lin + 0.5 * inter
            obuf[pl.ds(goff, 16)] = 1.0 / (1.0 + jnp.exp(-z))
            return gcarry

        lax.fori_loop(0, GROUPS, group_fn, 0)
        pltpu.sync_copy(obuf, out.at[pl.ds(row0, BC)])
        return carry

    lax.fori_loop(0, STEPS, step_fn, 0)


def kernel(x, emb_tables, weight_tables):
    mesh = plsc.VectorSubcoreMesh(core_axis_name="c", subcore_axis_name="s")

    src = jnp.transpose(emb_tables, (0, 2, 1)).reshape(F * K, V)
    tailsrc = jnp.pad(src[:, CPF * WC:], ((0, 0), (0, WC - TAIL)))
    tr = pl.kernel(
        _tr_body,
        out_type=jax.ShapeDtypeStruct((SC_ROWS * 128,), jnp.float32),
        mesh=mesh,
        compiler_params=pltpu.CompilerParams(needs_layout_passes=False),
        scratch_types=(
            [pltpu.VMEM((16, WC), jnp.float32) for _ in range(NB)]
            + [pltpu.VMEM((OBN,), jnp.float32) for _ in range(NB)]
            + [pltpu.SemaphoreType.DMA for _ in range(2 * NB)]
        ),
    )
    emb2d = tr(src, tailsrc).reshape(F * VP, K)

    w1d = weight_tables.reshape(F * V)
    pad = IDX_ROWS_PAD * 128 - IDX_PER_CHUNK
    fx = x + (jnp.arange(F, dtype=jnp.int32) * VP)[None, :]
    xw = fx.T.reshape(F, N_CHUNKS, BC).transpose(1, 0, 2)
    xw = xw.reshape(N_CHUNKS, IDX_PER_CHUNK)
    xw = jnp.pad(xw, ((0, 0), (0, pad))).reshape(N_CHUNKS, IDX_ROWS_PAD, 128)
    fm = pl.kernel(
        _fm_body,
        out_type=jax.ShapeDtypeStruct((B,), jnp.float32),
        mesh=mesh,
        compiler_params=pltpu.CompilerParams(use_tc_tiling_on_sc=False),
        scratch_types=[
            pltpu.VMEM((IDX_ROWS_PAD, 128), jnp.int32),
            pltpu.VMEM((IDX_PER_CHUNK, K), jnp.float32),
            pltpu.VMEM((IDX_PER_CHUNK,), jnp.float32),
            pltpu.VMEM((BC,), jnp.float32),
            pltpu.SemaphoreType.DMA,
        ],
    )
    return fm(xw, emb2d, w1d)


# speedup vs baseline: 1.0796x; 1.0796x over previous
import jax
import jax.numpy as jnp
from jax import lax
from jax.experimental import pallas as pl
from jax.experimental.pallas import tpu as pltpu
from jax.experimental.pallas import tpu_sc as plsc

F = 26
V = 100000
K = 16
B = 16384

NC = 2
NS = 16
NW = NC * NS

WC = 128
CPF = V // WC
TAIL = V - CPF * WC
RPF = 12500
VP = RPF * 8
SC_ROWS = F * RPF
NFULL = F * CPF
NB = 4
NITER = 159
OBN = WC * 16

ROWS_PER_W = B // NW
BC = 128
STEPS = ROWS_PER_W // BC
GROUPS = BC // 16
IDX_PER_CHUNK = BC * F
IDX_ROWS = IDX_PER_CHUNK // 128
IDX_ROWS_PAD = 32
N_CHUNKS = NW * STEPS


def _tr_body(src, tailsrc, out, *scr):
    slabs = scr[0:NB]
    obufs = scr[NB:2 * NB]
    semi = scr[2 * NB:3 * NB]
    semo = scr[3 * NB:4 * NB]
    cid = lax.axis_index("c")
    sid = lax.axis_index("s")
    w = cid * NS + sid
    lanes = lax.iota(jnp.int32, 16)

    def fc(j):
        u = w + NW * j
        return u, u // CPF, u % CPF

    def start_in(b, j):
        u, f, c = fc(j)

        @pl.when(u < NFULL)
        def _():
            pltpu.async_copy(
                src.at[pl.ds(f * 16, 16), pl.ds(c * WC, WC)],
                slabs[b], semi[b])

    def wait_in(b):
        pltpu.make_async_copy(
            src.at[pl.ds(0, 16), pl.ds(0, WC)], slabs[b], semi[b]).wait()

    lanes16 = lanes * 16
    idxr = [lanes16 + r for r in range(8)]

    def transpose_slab(b, ncols):
        for k in range(16):
            nch = ncols // 16
            vecs = [slabs[b][k, pl.ds(ch * 16, 16)] for ch in range(nch)]
            for ch in range(nch):
                off = ch * 256 + (k // 8) * 8
                plsc.store_scatter(
                    obufs[b].at[pl.ds(off, OBN - off)], [idxr[k % 8]],
                    vecs[ch])

    def start_out(b, j):
        u, f, c = fc(j)
        pltpu.async_copy(
            obufs[b], out.at[pl.ds((f * RPF + c * (WC // 8)) * 128, OBN)],
            semo[b])

    def wait_out(b):
        pltpu.make_async_copy(
            obufs[b], out.at[pl.ds(0, OBN)], semo[b]).wait()

    for b in range(NB):
        start_in(b, b)

    def ring(i, carry):
        for b in range(NB):
            j = NB * i + b
            u = w + NW * j

            @pl.when(jnp.logical_and(j >= NB, w + NW * (j - NB) < NFULL))
            def _():
                wait_out(b)

            @pl.when(u < NFULL)
            def _():
                wait_in(b)
                transpose_slab(b, WC)
                start_out(b, j)

            start_in(b, j + NB)
        return carry

    lax.fori_loop(0, NITER, ring, 0)

    for b in range(NB):
        jf = NB * (NITER - 1) + b

        @pl.when(w + NW * jf < NFULL)
        def _():
            wait_out(b)

    @pl.when(w < F)
    def _():
        pltpu.async_copy(
            tailsrc.at[pl.ds(w * 16, 16)], slabs[0], semi[0]).wait()
        transpose_slab(0, TAIL)
        pltpu.async_copy(
            obufs[0].at[pl.ds(0, TAIL * 16)],
            out.at[pl.ds((w * RPF + CPF * (WC // 8)) * 128, TAIL * 16)],
            semo[0]).wait()


def _fm_body(xw, emb, wtab, out, idx0, idx1, eb0, eb1, wb0, wb1, obuf,
             gs0, gs1, is0, is1):
    idxs = (idx0, idx1)
    ebufs = (eb0, eb1)
    wbufs = (wb0, wb1)
    gsems = (gs0, gs1)
    isems = (is0, is1)
    cid = lax.axis_index("c")
    sid = lax.axis_index("s")
    wid = cid * NS + sid
    cbase = wid * STEPS

    lanes = lax.iota(jnp.int32, 16)

    def issue_idx(p, step):
        pltpu.async_copy(xw.at[cbase + step], idxs[p], isems[p])

    def wait_idx(p):
        pltpu.make_async_copy(xw.at[0], idxs[p], isems[p]).wait()

    def issue_gathers(p):
        for j in range(IDX_ROWS):
            pltpu.async_copy(
                emb.at[idxs[p].at[j]], ebufs[p].at[pl.ds(j * 128, 128)],
                gsems[p])
            pltpu.async_copy(
                wtab.at[idxs[p].at[j]], wbufs[p].at[pl.ds(j * 128, 128)],
                gsems[p])

    def wait_gathers(p):
        pltpu.make_async_copy(
            emb.at[pl.ds(0, IDX_PER_CHUNK)], ebufs[p], gsems[p]).wait()
        pltpu.make_async_copy(
            wtab.at[pl.ds(0, IDX_PER_CHUNK)], wbufs[p], gsems[p]).wait()

    issue_idx(0, 0)
    wait_idx(0)
    issue_gathers(0)
    issue_idx(1, 1)

    def step_pair(i, carry):
        for p in range(2):
            step = 2 * i + p
            ebuf = ebufs[p]
            wbuf = wbufs[p]
            row0 = wid * ROWS_PER_W + step * BC
            wait_gathers(p)

            @pl.when(step + 1 < STEPS)
            def _():
                wait_idx(1 - p)
                issue_gathers(1 - p)

            @pl.when(step + 2 < STEPS)
            def _():
                issue_idx(p, step + 2)

            def group_fn(g, gcarry, ebuf=ebuf, wbuf=wbuf):
                goff = g * 16
                lin = wbuf[pl.ds(goff, 16)]
                for f in range(1, F):
                    lin = lin + wbuf[pl.ds(f * BC + goff, 16)]
                inter = jnp.zeros((16,), jnp.float32)
                for r16 in range(16):
                    roff = goff + r16
                    e = ebuf[roff, :]
                    s = e
                    q = e * e
                    for f in range(1, F):
                        e = ebuf[f * BC + roff, :]
                        s = s + e
                        q = q + e * e
                    t = s * s - q
                    tot = t[0]
                    for li in range(1, 16):
                        tot = tot + t[li]
                    inter = jnp.where(lanes == r16, tot, inter)
                z = lin + 0.5 * inter
                obuf[pl.ds(goff, 16)] = 1.0 / (1.0 + jnp.exp(-z))
                return gcarry

            lax.fori_loop(0, GROUPS, group_fn, 0)
            pltpu.sync_copy(obuf, out.at[pl.ds(row0, BC)])
        return carry

    lax.fori_loop(0, STEPS // 2, step_pair, 0)


def kernel(x, emb_tables, weight_tables):
    mesh = plsc.VectorSubcoreMesh(core_axis_name="c", subcore_axis_name="s")

    src = jnp.transpose(emb_tables, (0, 2, 1)).reshape(F * K, V)
    tailsrc = jnp.pad(src[:, CPF * WC:], ((0, 0), (0, WC - TAIL)))
    tr = pl.kernel(
        _tr_body,
        out_type=jax.ShapeDtypeStruct((SC_ROWS * 128,), jnp.float32),
        mesh=mesh,
        compiler_params=pltpu.CompilerParams(needs_layout_passes=False),
        scratch_types=(
            [pltpu.VMEM((16, WC), jnp.float32) for _ in range(NB)]
            + [pltpu.VMEM((OBN,), jnp.float32) for _ in range(NB)]
            + [pltpu.SemaphoreType.DMA for _ in range(2 * NB)]
        ),
    )
    emb2d = tr(src, tailsrc).reshape(F * VP, K)

    w1d = weight_tables.reshape(F * V)
    pad = IDX_ROWS_PAD * 128 - IDX_PER_CHUNK
    fx = x + (jnp.arange(F, dtype=jnp.int32) * VP)[None, :]
    xw = fx.T.reshape(F, N_CHUNKS, BC).transpose(1, 0, 2)
    xw = xw.reshape(N_CHUNKS, IDX_PER_CHUNK)
    xw = jnp.pad(xw, ((0, 0), (0, pad))).reshape(N_CHUNKS, IDX_ROWS_PAD, 128)
    fm = pl.kernel(
        _fm_body,
        out_type=jax.ShapeDtypeStruct((B,), jnp.float32),
        mesh=mesh,
        compiler_params=pltpu.CompilerParams(use_tc_tiling_on_sc=False),
        scratch_types=[
            pltpu.VMEM((IDX_ROWS_PAD, 128), jnp.int32),
            pltpu.VMEM((IDX_ROWS_PAD, 128), jnp.int32),
            pltpu.VMEM((IDX_PER_CHUNK, K), jnp.float32),
            pltpu.VMEM((IDX_PER_CHUNK, K), jnp.float32),
            pltpu.VMEM((IDX_PER_CHUNK,), jnp.float32),
            pltpu.VMEM((IDX_PER_CHUNK,), jnp.float32),
            pltpu.VMEM((BC,), jnp.float32),
            pltpu.SemaphoreType.DMA,
            pltpu.SemaphoreType.DMA,
            pltpu.SemaphoreType.DMA,
            pltpu.SemaphoreType.DMA,
        ],
    )
    return fm(xw, emb2d, w1d)
